# Initial kernel scaffold; baseline (speedup 1.0000x reference)
import jax
import jax.numpy as jnp
from jax.experimental import pallas as pl


def kernel(periph_seq, imu_seq, traj_seq, W1, b1, W2, b2):
    raise NotImplementedError("write your pallas kernel here")


# speedup vs baseline: 4.7428x; 4.7428x over previous
import jax
import jax.numpy as jnp
from jax.experimental import pallas as pl
from jax.experimental.pallas import tpu as pltpu

T, B = 32, 64
P, I, R = 1024, 64, 128
H, N = 64, 4096
K = 8


def _body(periph_ref, imu_ref, traj_ref, w1_ref, b1_ref, w2_ref, b2_ref,
          scores_ref, gate_ref, acc_p, acc_i, acc_t):
    t = pl.program_id(0)

    @pl.when(t == 0)
    def _():
        acc_p[...] = jnp.zeros_like(acc_p)
        acc_i[...] = jnp.zeros_like(acc_i)
        acc_t[...] = jnp.zeros_like(acc_t)

    acc_p[...] += periph_ref[0]
    acc_i[...] += imu_ref[0]
    acc_t[...] += traj_ref[0]

    @pl.when(t == T - 1)
    def _():
        inv = jnp.float32(1.0 / T)
        xp = acc_p[...] * inv
        xi = acc_i[...] * inv
        xt = acc_t[...] * inv
        w1 = w1_ref[...]
        h = (jnp.dot(xp, w1[0:P], preferred_element_type=jnp.float32)
             + jnp.dot(xi, w1[P:P + I], preferred_element_type=jnp.float32)
             + jnp.dot(xt, w1[P + I:P + I + R],
                       preferred_element_type=jnp.float32)
             + b1_ref[...])
        h = jnp.maximum(h, 0.0)
        scores = (jnp.dot(h, w2_ref[...], preferred_element_type=jnp.float32)
                  + b2_ref[...])
        scores_ref[...] = scores

        neg = jnp.float32(-jnp.inf)
        cur = jnp.full((B, 1), jnp.inf, jnp.float32)
        remaining = jnp.full((B, 1), K, jnp.int32)
        thresh = jnp.full((B, 1), neg, jnp.float32)
        for _ in range(K):
            masked = jnp.where(scores < cur, scores, neg)
            m = jnp.max(masked, axis=1, keepdims=True)
            n = jnp.sum((scores == m).astype(jnp.int32), axis=1,
                        keepdims=True)
            take = remaining > 0
            thresh = jnp.where(take, m, thresh)
            remaining = jnp.where(take, remaining - n, remaining)
            cur = m
        gate_ref[...] = (scores >= thresh).astype(jnp.float32)


def kernel(periph_seq, imu_seq, traj_seq, W1, b1, W2, b2):
    scores, gate = pl.pallas_call(
        _body,
        grid=(T,),
        in_specs=[
            pl.BlockSpec((1, B, P), lambda t: (t, 0, 0)),
            pl.BlockSpec((1, B, I), lambda t: (t, 0, 0)),
            pl.BlockSpec((1, B, R), lambda t: (t, 0, 0)),
            pl.BlockSpec((P + I + R, H), lambda t: (0, 0)),
            pl.BlockSpec((1, H), lambda t: (0, 0)),
            pl.BlockSpec((H, N), lambda t: (0, 0)),
            pl.BlockSpec((1, N), lambda t: (0, 0)),
        ],
        out_specs=[pl.BlockSpec((B, N), lambda t: (0, 0)),
                   pl.BlockSpec((B, N), lambda t: (0, 0))],
        out_shape=[jax.ShapeDtypeStruct((B, N), jnp.float32),
                   jax.ShapeDtypeStruct((B, N), jnp.float32)],
        scratch_shapes=[pltpu.VMEM((B, P), jnp.float32),
                        pltpu.VMEM((B, I), jnp.float32),
                        pltpu.VMEM((B, R), jnp.float32)],
    )(periph_seq, imu_seq, traj_seq, W1, b1.reshape(1, H), W2,
      b2.reshape(1, N))
    return (scores, gate)
